# baseline (speedup 1.0000x reference)
import functools

import jax
import jax.numpy as jnp
from jax import lax
from jax.experimental import pallas as pl
from jax.experimental.pallas import tpu as pltpu


def _conv_stats_kernel(x0_ref, x1_ref, x2_ref, w_ref, y_ref, s_ref, *,
                       kh, kw, ho, wo, cin):
    hw = ho * wo
    cout = y_ref.shape[0]
    acc = jnp.zeros((cout, hw), jnp.float32)
    for dt, xr in enumerate((x0_ref, x1_ref, x2_ref)):
        for dh in range(kh):
            for dw in range(kw):
                p = xr[dh:dh + ho, dw:dw + wo, :]
                p = p.reshape(hw, cin).astype(jnp.bfloat16)
                acc = acc + lax.dot_general(
                    w_ref[dt * kh * kw + dh * kw + dw], p,
                    (((0,), (1,)), ((), ())),
                    preferred_element_type=jnp.float32)
    y_ref[...] = acc.astype(jnp.bfloat16)
    s_ref[:, 0:1] = jnp.sum(acc, axis=1, keepdims=True)
    s_ref[:, 1:2] = jnp.sum(acc * acc, axis=1, keepdims=True)


def _bn_relu_kernel(y_ref, sc_ref, sh_ref, o_ref):
    y = y_ref[...].astype(jnp.float32)
    o_ref[...] = jnp.maximum(y * sc_ref[...] + sh_ref[...], 0.0)


def kernel(x_pt, w_pt, gamma, beta):
    n, cin, t, h, w = x_pt.shape
    cout = w_pt.shape[0]
    kt, kh, kw = w_pt.shape[2:]
    hw = h * w
    eps = 1e-5

    xb = jnp.pad(jnp.transpose(x_pt, (0, 2, 3, 4, 1)).astype(jnp.float32),
                 [(0, 0), (1, 1), (1, 1), (1, 1), (0, 0)])
    wt = jnp.transpose(w_pt, (2, 3, 4, 1, 0)).reshape(kt * kh * kw, cin, cout)
    wt = wt.astype(jnp.bfloat16)
    hp, wp = h + 2, w + 2

    y, stats = pl.pallas_call(
        functools.partial(_conv_stats_kernel, kh=kh, kw=kw, ho=h, wo=w,
                          cin=cin),
        grid=(n, t),
        in_specs=[
            pl.BlockSpec((None, None, hp, wp, cin),
                         lambda b, tt: (b, tt, 0, 0, 0)),
            pl.BlockSpec((None, None, hp, wp, cin),
                         lambda b, tt: (b, tt + 1, 0, 0, 0)),
            pl.BlockSpec((None, None, hp, wp, cin),
                         lambda b, tt: (b, tt + 2, 0, 0, 0)),
            pl.BlockSpec((kt * kh * kw, cin, cout),
                         lambda b, tt: (0, 0, 0)),
        ],
        out_specs=(
            pl.BlockSpec((None, None, cout, hw), lambda b, tt: (b, tt, 0, 0)),
            pl.BlockSpec((None, None, cout, 2), lambda b, tt: (b, tt, 0, 0)),
        ),
        out_shape=(
            jax.ShapeDtypeStruct((n, t, cout, hw), jnp.bfloat16),
            jax.ShapeDtypeStruct((n, t, cout, 2), jnp.float32),
        ),
        compiler_params=pltpu.CompilerParams(
            dimension_semantics=("parallel", "parallel"),
            vmem_limit_bytes=48 * 1024 * 1024,
        ),
    )(xb, xb, xb, wt)

    s = jnp.sum(stats, axis=(0, 1))
    count = float(n * t * h * w)
    mean = s[:, 0] / count
    var = jnp.maximum(s[:, 1] / count - mean * mean, 0.0)
    scale = gamma.astype(jnp.float32) * lax.rsqrt(var + eps)
    shift = beta.astype(jnp.float32) - mean * scale
    sc = scale.reshape(cout, 1)
    sh = shift.reshape(cout, 1)

    yt = jnp.transpose(y, (0, 2, 1, 3)).reshape(n, cout, t * hw)

    tf = t
    for cand in range(1, t + 1):
        if t % cand == 0 and (cand * hw) % 128 == 0:
            tf = cand
            break
    chunk = tf * hw

    out = pl.pallas_call(
        _bn_relu_kernel,
        grid=(n, t // tf),
        in_specs=[
            pl.BlockSpec((None, cout, chunk), lambda b, k: (b, 0, k)),
            pl.BlockSpec((cout, 1), lambda b, k: (0, 0)),
            pl.BlockSpec((cout, 1), lambda b, k: (0, 0)),
        ],
        out_specs=pl.BlockSpec((None, cout, chunk), lambda b, k: (b, 0, k)),
        out_shape=jax.ShapeDtypeStruct((n, cout, t * hw), jnp.float32),
        compiler_params=pltpu.CompilerParams(
            dimension_semantics=("parallel", "parallel"),
            vmem_limit_bytes=48 * 1024 * 1024,
        ),
    )(yt, sc, sh)

    return out.reshape(n, cout, t, h, w)
